# baseline (speedup 1.0000x reference)
import functools

import jax
import jax.numpy as jnp
from jax.experimental import pallas as pl
from jax.experimental.pallas import tpu as pltpu

NEG_INF = float("-inf")


def _gelu(z):
    return z * 0.5 * (1.0 + jax.lax.erf(z * 0.7071067811865476))


def _main_body(eid_ref, pperm_ref, h_ref,
               wc1_ref, bc1_ref, wc2_ref, bc2_ref,
               wf1_ref, bf1_ref, wf2_ref, bf2_ref,
               sc_out, sf_out,
               hbuf, sem, acc_c, acc_f,
               *, bt, n_g, n_h, chunk):
    g = pl.program_id(0)
    hs = pl.program_id(1)

    def copy_rows(tile, lo, hi, slot, do_start):
        def body_fn(j, carry):
            row = pperm_ref[tile * bt + j]
            cp = pltpu.make_async_copy(
                h_ref.at[pl.ds(row, 1)], hbuf.at[slot, pl.ds(j, 1)], sem)
            if do_start:
                cp.start()
            else:
                cp.wait()
            return carry
        jax.lax.fori_loop(lo, hi, body_fn, 0)

    @pl.when((g == 0) & (hs == 0))
    def _():
        copy_rows(0, 0, bt, 0, True)

    @pl.when(hs == 0)
    def _():
        copy_rows(g, 0, bt, jax.lax.rem(g, 3), False)

    @pl.when(g + 1 < n_g)
    def _():
        copy_rows(g + 1, hs * chunk, (hs + 1) * chunk,
                  jax.lax.rem(g + 1, 3), True)

    slot = jax.lax.rem(g, 3)
    hh = hbuf[pl.ds(slot, 1)][0].astype(jnp.bfloat16)

    zc = jax.lax.dot_general(
        hh, wc1_ref[...], (((1,), (1,)), ((), ())),
        preferred_element_type=jnp.float32)
    zc = _gelu(zc + bc1_ref[...]).astype(jnp.bfloat16)
    pc = jax.lax.dot_general(
        zc, wc2_ref[...], (((1,), (1,)), ((), ())),
        preferred_element_type=jnp.float32)

    zf = jax.lax.dot_general(
        hh, wf1_ref[0], (((1,), (1,)), ((), ())),
        preferred_element_type=jnp.float32)
    zf = _gelu(zf + bf1_ref[0]).astype(jnp.bfloat16)
    pf = jax.lax.dot_general(
        zf, wf2_ref[0], (((1,), (1,)), ((), ())),
        preferred_element_type=jnp.float32)

    @pl.when(hs == 0)
    def _():
        acc_c[...] = pc
        acc_f[...] = pf

    @pl.when(hs != 0)
    def _():
        acc_c[...] += pc
        acc_f[...] += pf

    @pl.when(hs == n_h - 1)
    def _():
        sc_out[...] = (acc_c[...] + bc2_ref[...]).astype(jnp.bfloat16)
        sf_out[...] = (acc_f[...] + bf2_ref[0]).astype(jnp.bfloat16)


def _epi_body(invpos_ref, labels_ref, sc_ref, sf_ref,
              coarse_out, fine_out, flat_out, *, gbt, nf0, nf1):
    ip = invpos_ref[...]
    iota = jax.lax.broadcasted_iota(jnp.int32, (ip.shape[0], gbt), 1)
    pmat = (ip == iota).astype(jnp.bfloat16)
    coarse_out[...] = jax.lax.dot_general(
        pmat, sc_ref[...], (((1,), (0,)), ((), ())),
        preferred_element_type=jnp.float32)
    logits = jax.lax.dot_general(
        pmat, sf_ref[...], (((1,), (0,)), ((), ())),
        preferred_element_type=jnp.float32)
    mask = labels_ref[...] == 0
    neg = jnp.float32(NEG_INF)
    col = jax.lax.broadcasted_iota(jnp.int32, logits.shape, 1)
    fine_out[...] = jnp.where(mask & (col >= nf0), neg, logits)
    flat_out[...] = jnp.concatenate(
        [jnp.where(mask, logits[:, :nf0], neg),
         jnp.where(mask, neg, logits)], axis=1)


def kernel(h, coarse_labels, Wc1, bc1, Wc2, bc2,
           Wf0_1, bf0_1, Wf0_2, bf0_2, Wf1_1, bf1_1, Wf1_2, bf1_2):
    B, IN = h.shape
    H = Wc1.shape[0]
    NC = Wc2.shape[0]
    NF0 = Wf0_2.shape[0]
    NF1 = Wf1_2.shape[0]
    bt = min(512, B)
    hc = min(512, H)
    n_h = H // hc
    n_g = B // bt + 1
    gbt = n_g * bt
    chunk = bt // n_h

    labels = coarse_labels.astype(jnp.int32)
    m0 = labels == 0
    c0 = jnp.cumsum(m0.astype(jnp.int32))
    count0 = c0[-1]
    c0ex = c0 - m0.astype(jnp.int32)
    idx = jnp.arange(B, dtype=jnp.int32)
    t1 = (count0 + bt - 1) // bt
    pos = jnp.where(m0, c0ex, t1 * bt + idx - c0ex)
    pperm = jnp.zeros((gbt,), jnp.int32).at[pos].set(idx)
    eid = (jnp.arange(n_g, dtype=jnp.int32) >= t1).astype(jnp.int32)
    invpos = pos.reshape(B, 1)
    labels2 = labels.reshape(B, 1)

    bf = jnp.bfloat16
    wc1b = Wc1.astype(bf)
    bc1r = bc1.reshape(1, H)
    wc2b = Wc2.astype(bf)
    bc2r = bc2.reshape(1, NC)
    wf1_st = jnp.stack([Wf0_1, Wf1_1]).astype(bf)
    bf1_st = jnp.stack([bf0_1, bf1_1]).reshape(2, 1, H)
    wf2_st = jnp.stack([
        jnp.concatenate([Wf0_2, jnp.zeros((NF1 - NF0, H), Wf0_2.dtype)], 0),
        Wf1_2]).astype(bf)
    bf2_st = jnp.stack([
        jnp.concatenate([bf0_2, jnp.zeros((NF1 - NF0,), bf0_2.dtype)]),
        bf1_2]).reshape(2, 1, NF1)

    grid_spec = pltpu.PrefetchScalarGridSpec(
        num_scalar_prefetch=2,
        grid=(n_g, n_h),
        in_specs=[
            pl.BlockSpec(memory_space=pl.ANY),
            pl.BlockSpec((hc, IN), lambda g, hs, e, p: (hs, 0)),
            pl.BlockSpec((1, hc), lambda g, hs, e, p: (0, hs)),
            pl.BlockSpec((NC, hc), lambda g, hs, e, p: (0, hs)),
            pl.BlockSpec((1, NC), lambda g, hs, e, p: (0, 0)),
            pl.BlockSpec((1, hc, IN), lambda g, hs, e, p: (e[g], hs, 0)),
            pl.BlockSpec((1, 1, hc), lambda g, hs, e, p: (e[g], 0, hs)),
            pl.BlockSpec((1, NF1, hc), lambda g, hs, e, p: (e[g], 0, hs)),
            pl.BlockSpec((1, 1, NF1), lambda g, hs, e, p: (e[g], 0, 0)),
        ],
        out_specs=[
            pl.BlockSpec((bt, NC), lambda g, hs, e, p: (g, 0)),
            pl.BlockSpec((bt, NF1), lambda g, hs, e, p: (g, 0)),
        ],
        scratch_shapes=[
            pltpu.VMEM((3, bt, IN), jnp.float32),
            pltpu.SemaphoreType.DMA,
            pltpu.VMEM((bt, NC), jnp.float32),
            pltpu.VMEM((bt, NF1), jnp.float32),
        ],
    )
    sc, sf = pl.pallas_call(
        functools.partial(_main_body, bt=bt, n_g=n_g, n_h=n_h, chunk=chunk),
        grid_spec=grid_spec,
        out_shape=[
            jax.ShapeDtypeStruct((gbt, NC), bf),
            jax.ShapeDtypeStruct((gbt, NF1), bf),
        ],
    )(eid, pperm, h, wc1b, bc1r, wc2b, bc2r, wf1_st, bf1_st, wf2_st, bf2_st)

    bt2 = min(512, B)
    coarse, fine, flat = pl.pallas_call(
        functools.partial(_epi_body, gbt=gbt, nf0=NF0, nf1=NF1),
        grid=(B // bt2,),
        in_specs=[
            pl.BlockSpec((bt2, 1), lambda b: (b, 0)),
            pl.BlockSpec((bt2, 1), lambda b: (b, 0)),
            pl.BlockSpec((gbt, NC), lambda b: (0, 0)),
            pl.BlockSpec((gbt, NF1), lambda b: (0, 0)),
        ],
        out_specs=[
            pl.BlockSpec((bt2, NC), lambda b: (b, 0)),
            pl.BlockSpec((bt2, NF1), lambda b: (b, 0)),
            pl.BlockSpec((bt2, NF0 + NF1), lambda b: (b, 0)),
        ],
        out_shape=[
            jax.ShapeDtypeStruct((B, NC), jnp.float32),
            jax.ShapeDtypeStruct((B, NF1), jnp.float32),
            jax.ShapeDtypeStruct((B, NF0 + NF1), jnp.float32),
        ],
    )(invpos, labels2, sc, sf)
    return (coarse, fine, flat)
